# Initial kernel scaffold; baseline (speedup 1.0000x reference)
import jax
import jax.numpy as jnp
from jax.experimental import pallas as pl


def kernel(state, edge_index, Wq, bq, Wk, bk, Wv, bv):
    raise NotImplementedError("write your pallas kernel here")


# speedup vs baseline: 14.5179x; 14.5179x over previous
import functools

import jax
import jax.numpy as jnp
from jax import lax
from jax.experimental import pallas as pl
from jax.experimental.pallas import tpu as pltpu
from jax.experimental.pallas import tpu_sc as plsc

N = 10000
E = 320000
IN_DIM = 128
H = 8
D = 16
HD = H * D

NC = 2
NS = 16
NW = NC * NS
EPW = E // NW
C = 80
G = C // 16
NCHUNK = EPW // C
ROWW = 144
RPT = N // NS


def _proj_body(x_ref, wq_ref, bq_ref, wk_ref, bk_ref, wv_ref, bv_ref,
               q_ref, k_ref, v_ref):
    x = x_ref[...]
    scale = 1.0 / (D ** 0.5)
    q = lax.dot_general(x, wq_ref[...], (((1,), (0,)), ((), ())),
                        preferred_element_type=jnp.float32)
    k = lax.dot_general(x, wk_ref[...], (((1,), (0,)), ((), ())),
                        preferred_element_type=jnp.float32)
    v = lax.dot_general(x, wv_ref[...], (((1,), (0,)), ((), ())),
                        preferred_element_type=jnp.float32)
    q_ref[...] = (q + bq_ref[...]) * scale
    k_ref[...] = k + bk_ref[...]
    v_ref[...] = v + bv_ref[...]


def _project(state, Wq, bq, Wk, bk, Wv, bv):
    blk = 1000
    grid = (N // blk,)
    w_spec = pl.BlockSpec((IN_DIM, HD), lambda i: (0, 0))
    b_spec = pl.BlockSpec((1, HD), lambda i: (0, 0))
    row_spec = pl.BlockSpec((blk, IN_DIM), lambda i: (i, 0))
    return pl.pallas_call(
        _proj_body,
        grid=grid,
        in_specs=[row_spec, w_spec, b_spec, w_spec, b_spec, w_spec, b_spec],
        out_specs=[row_spec, row_spec, row_spec],
        out_shape=[jax.ShapeDtypeStruct((N, HD), jnp.float32)] * 3,
    )(state, Wq, bq.reshape(1, HD), Wk, bk.reshape(1, HD),
      Wv, bv.reshape(1, HD))


def _edge_body(k_hbm, q_hbm, v_hbm, src_hbm, dst_hbm, out_hbm,
               idx_src, idx_dst, kg, qg, vg, wrow, zbuf, acc,
               sem_k, sem_q, sem_v):
    cid = lax.axis_index("c")
    sid = lax.axis_index("s")
    wid = cid * NS + sid
    liota = lax.iota(jnp.int32, 16)
    zeros16 = jnp.zeros((16,), jnp.float32)

    def _zfill(j, _):
        zbuf[j // 9, pl.ds((j % 9) * 16, 16)] = zeros16
        return 0
    lax.fori_loop(0, 25 * 9, _zfill, 0)

    def _zcopy(t, _):
        pltpu.sync_copy(zbuf, acc.at[pl.ds(sid * RPT + t * 25, 25)])
        return 0
    lax.fori_loop(0, RPT // 25, _zcopy, 0)

    for j in range(8):
        plsc.store_scatter(wrow, [liota, jnp.full((16,), 136 + j, jnp.int32)],
                           zeros16)

    plsc.subcore_barrier()

    ebase = wid * EPW

    def chunk_body(i, _):
        base = ebase + i * C
        pltpu.sync_copy(src_hbm.at[pl.ds(base, C)], idx_src)
        pltpu.sync_copy(dst_hbm.at[pl.ds(base, C)], idx_dst)
        cp_k = pltpu.async_copy(k_hbm.at[idx_src], kg, sem_k)
        cp_q = pltpu.async_copy(q_hbm.at[idx_dst], qg, sem_q)
        cp_v = pltpu.async_copy(v_hbm.at[idx_src], vg, sem_v)
        cp_k.wait()
        cp_q.wait()
        cp_v.wait()

        def group_body(g, _):
            rows = g * 16 + liota
            for h in range(H):
                s_acc = zeros16
                for d in range(D):
                    col = jnp.full((16,), h * D + d, jnp.int32)
                    kvv = plsc.load_gather(kg, [rows, col])
                    qvv = plsc.load_gather(qg, [rows, col])
                    s_acc = s_acc + kvv * qvv
                s = jnp.exp(jnp.clip(s_acc, -5.0, 5.0))
                for d in range(D):
                    col = jnp.full((16,), h * D + d, jnp.int32)
                    vvv = plsc.load_gather(vg, [rows, col])
                    plsc.store_scatter(wrow, [liota, col], vvv * s)
                plsc.store_scatter(
                    wrow, [liota, jnp.full((16,), HD + h, jnp.int32)], s)
            dvec = idx_dst[pl.ds(g * 16, 16)]
            pltpu.sync_copy(wrow, acc.at[dvec], add=True)
            return 0

        lax.fori_loop(0, G, group_body, 0)
        return 0

    lax.fori_loop(0, NCHUNK, chunk_body, 0)

    plsc.subcore_barrier()

    pltpu.sync_copy(acc.at[pl.ds(sid * RPT, RPT)],
                    out_hbm.at[cid, pl.ds(sid * RPT, RPT)])


def _edge_phase(k, q, v, src, dst):
    mesh = plsc.VectorSubcoreMesh(core_axis_name="c", subcore_axis_name="s")
    kern = pl.kernel(
        _edge_body,
        out_type=jax.ShapeDtypeStruct((NC, N, ROWW), jnp.float32),
        mesh=mesh,
        scratch_types=[
            pltpu.VMEM((C,), jnp.int32),
            pltpu.VMEM((C,), jnp.int32),
            pltpu.VMEM((C, HD), jnp.float32),
            pltpu.VMEM((C, HD), jnp.float32),
            pltpu.VMEM((C, HD), jnp.float32),
            pltpu.VMEM((16, ROWW), jnp.float32),
            pltpu.VMEM((25, ROWW), jnp.float32),
            pltpu.VMEM_SHARED((N, ROWW), jnp.float32),
            pltpu.SemaphoreType.DMA,
            pltpu.SemaphoreType.DMA,
            pltpu.SemaphoreType.DMA,
        ],
        compiler_params=pltpu.CompilerParams(use_tc_tiling_on_sc=False,
                                             needs_layout_passes=False),
    )
    return kern(k, q, v, src, dst)


def _combine_body(p_ref, out_ref):
    a = p_ref[0] + p_ref[1]
    cols = []
    for h in range(H):
        wv = a[:, h * D:(h + 1) * D]
        z = a[:, HD + h:HD + h + 1]
        cols.append(wv / z)
    out_ref[...] = jnp.concatenate(cols, axis=1)


def _combine(parts):
    blk = 1000
    return pl.pallas_call(
        _combine_body,
        grid=(N // blk,),
        in_specs=[pl.BlockSpec((NC, blk, ROWW), lambda i: (0, i, 0))],
        out_specs=pl.BlockSpec((blk, HD), lambda i: (i, 0)),
        out_shape=jax.ShapeDtypeStruct((N, HD), jnp.float32),
    )(parts)


def kernel(state, edge_index, Wq, bq, Wk, bk, Wv, bv):
    q, k, v = _project(state, Wq, bq, Wk, bk, Wv, bv)
    src = edge_index[0]
    dst = edge_index[1]
    parts = _edge_phase(k, q, v, src, dst)
    out = _combine(parts)
    return out.reshape(N, H, D)


# speedup vs baseline: 15.9182x; 1.0965x over previous
import jax
import jax.numpy as jnp
from jax import lax
from jax.experimental import pallas as pl
from jax.experimental.pallas import tpu as pltpu
from jax.experimental.pallas import tpu_sc as plsc

N = 10000
E = 320000
IN_DIM = 128
H = 8
D = 16
HD = H * D
HH = H // 2
HW = HH * D

NC = 2
NS = 16
EPT = E // NS
C = 80
G = C // 16
NCHUNK = EPT // C
NPAIR = NCHUNK // 2
ROWW = 80
RPT = N // NS


def _proj_body(x_ref, wq_ref, bq_ref, wk_ref, bk_ref, wv_ref, bv_ref,
               qt_ref, kvt_ref):
    x = x_ref[...]
    scale = 1.0 / (D ** 0.5)
    q = lax.dot_general(x, wq_ref[...], (((1,), (0,)), ((), ())),
                        preferred_element_type=jnp.float32)
    k = lax.dot_general(x, wk_ref[...], (((1,), (0,)), ((), ())),
                        preferred_element_type=jnp.float32)
    v = lax.dot_general(x, wv_ref[...], (((1,), (0,)), ((), ())),
                        preferred_element_type=jnp.float32)
    q = (q + bq_ref[...]) * scale
    k = k + bk_ref[...]
    v = v + bv_ref[...]
    for c in range(NC):
        qt_ref[c] = q[:, c * HW:(c + 1) * HW]
        kvt_ref[c, :, :HW] = k[:, c * HW:(c + 1) * HW]
        kvt_ref[c, :, HW:] = v[:, c * HW:(c + 1) * HW]


def _project(state, Wq, bq, Wk, bk, Wv, bv):
    blk = 1000
    grid = (N // blk,)
    w_spec = pl.BlockSpec((IN_DIM, HD), lambda i: (0, 0))
    b_spec = pl.BlockSpec((1, HD), lambda i: (0, 0))
    x_spec = pl.BlockSpec((blk, IN_DIM), lambda i: (i, 0))
    qt, kvt = pl.pallas_call(
        _proj_body,
        grid=grid,
        in_specs=[x_spec, w_spec, b_spec, w_spec, b_spec, w_spec, b_spec],
        out_specs=[pl.BlockSpec((NC, blk, HW), lambda i: (0, i, 0)),
                   pl.BlockSpec((NC, blk, 2 * HW), lambda i: (0, i, 0))],
        out_shape=[jax.ShapeDtypeStruct((NC, N, HW), jnp.float32),
                   jax.ShapeDtypeStruct((NC, N, 2 * HW), jnp.float32)],
    )(state, Wq, bq.reshape(1, HD), Wk, bk.reshape(1, HD),
      Wv, bv.reshape(1, HD))
    return qt.reshape(NC * N, HW), kvt.reshape(NC * N, 2 * HW)


def _edge_body(kv_hbm, q_hbm, src_hbm, dst_hbm, out_hbm,
               isA, idA, iqA, isB, idB, iqB,
               kvgA, qgA, kvgB, qgB, wrow3, acc,
               sem_iA, sem_iB, sem_gA, sem_gB, sem_sc):
    cid = lax.axis_index("c")
    sid = lax.axis_index("s")
    cbase = cid * N
    liota = lax.iota(jnp.int32, 16)
    zeros16 = jnp.zeros((16,), jnp.float32)

    def _zfill(j, _):
        wrow3[j // (16 * G), (j % (16 * G)) // G, pl.ds((j % G) * 16, 16)] = (
            zeros16)
        return 0
    lax.fori_loop(0, G * 16 * G, _zfill, 0)

    def _zcopy(t, _):
        pltpu.sync_copy(wrow3.at[0], acc.at[pl.ds(sid * RPT + t * 16, 16)])
        return 0
    lax.fori_loop(0, RPT // 16, _zcopy, 0)
    pltpu.sync_copy(wrow3.at[0, pl.ds(0, 1)],
                    acc.at[pl.ds(sid * RPT + (RPT // 16) * 16, 1)])

    plsc.subcore_barrier()

    ebase = sid * EPT

    def issue_idx(c, is_buf, id_buf, sem):
        base = ebase + c * C
        pltpu.async_copy(src_hbm.at[pl.ds(base, C)], is_buf, sem)
        pltpu.async_copy(dst_hbm.at[pl.ds(base, C)], id_buf, sem)

    def drain_idx(sem, is_buf, id_buf):
        pltpu.make_async_copy(src_hbm.at[pl.ds(0, C)], is_buf, sem).wait()
        pltpu.make_async_copy(src_hbm.at[pl.ds(0, C)], id_buf, sem).wait()

    def adjust_idx(is_buf, id_buf, iq_buf):
        for j in range(G):
            sl = pl.ds(j * 16, 16)
            is_buf[sl] = is_buf[sl] + cbase
            iq_buf[sl] = id_buf[sl] + cbase

    def issue_gath(is_buf, iq_buf, kvg, qg, sem):
        pltpu.async_copy(kv_hbm.at[is_buf], kvg, sem)
        pltpu.async_copy(q_hbm.at[iq_buf], qg, sem)

    def drain_gath(sem, kvg, qg):
        pltpu.make_async_copy(kv_hbm.at[pl.ds(0, C)], kvg, sem).wait()
        pltpu.make_async_copy(q_hbm.at[pl.ds(0, C)], qg, sem).wait()

    def comp(kvg, qg, id_buf):
        def group_body(g, _):
            rows = g * 16 + liota
            gv = jnp.full((16,), g, jnp.int32)
            for h in range(HH):
                s_acc = zeros16
                for d in range(D):
                    col = jnp.full((16,), h * D + d, jnp.int32)
                    kvv = plsc.load_gather(kvg, [rows, col])
                    qvv = plsc.load_gather(qg, [rows, col])
                    s_acc = s_acc + kvv * qvv
                s = jnp.exp(jnp.clip(s_acc, -5.0, 5.0))
                for d in range(D):
                    col = jnp.full((16,), h * D + d, jnp.int32)
                    vvv = plsc.load_gather(
                        kvg, [rows, jnp.full((16,), HW + h * D + d,
                                             jnp.int32)])
                    plsc.store_scatter(wrow3, [gv, liota, col], vvv * s)
                plsc.store_scatter(
                    wrow3, [gv, liota, jnp.full((16,), HW + h, jnp.int32)], s)
            dvec = id_buf[pl.ds(g * 16, 16)]
            pltpu.async_copy(wrow3.at[g], acc.at[dvec], sem_sc, add=True)
            return 0

        lax.fori_loop(0, G, group_body, 0)
        for _ in range(G):
            pltpu.make_async_copy(out_hbm.at[0, pl.ds(0, 16)],
                                  wrow3.at[0], sem_sc).wait()

    issue_idx(0, isA, idA, sem_iA)
    drain_idx(sem_iA, isA, idA)
    adjust_idx(isA, idA, iqA)
    issue_gath(isA, iqA, kvgA, qgA, sem_gA)
    issue_idx(1, isB, idB, sem_iB)

    def pair_body(t, _):
        a = 2 * t
        b = 2 * t + 1
        drain_gath(sem_gA, kvgA, qgA)
        drain_idx(sem_iB, isB, idB)
        adjust_idx(isB, idB, iqB)
        issue_gath(isB, iqB, kvgB, qgB, sem_gB)
        comp(kvgA, qgA, idA)

        @pl.when(t < NPAIR - 1)
        def _():
            issue_idx(a + 2, isA, idA, sem_iA)

        drain_gath(sem_gB, kvgB, qgB)

        @pl.when(t < NPAIR - 1)
        def _():
            drain_idx(sem_iA, isA, idA)
            adjust_idx(isA, idA, iqA)
            issue_gath(isA, iqA, kvgA, qgA, sem_gA)

        comp(kvgB, qgB, idB)

        @pl.when(t < NPAIR - 1)
        def _():
            issue_idx(b + 2, isB, idB, sem_iB)

        return 0

    lax.fori_loop(0, NPAIR, pair_body, 0)

    plsc.subcore_barrier()

    pltpu.sync_copy(acc.at[pl.ds(sid * RPT, RPT)],
                    out_hbm.at[cid, pl.ds(sid * RPT, RPT)])


def _edge_phase(kv, q, src, dst):
    mesh = plsc.VectorSubcoreMesh(core_axis_name="c", subcore_axis_name="s")
    kern = pl.kernel(
        _edge_body,
        out_type=jax.ShapeDtypeStruct((NC, N, ROWW), jnp.float32),
        mesh=mesh,
        scratch_types=[
            pltpu.VMEM((C,), jnp.int32),
            pltpu.VMEM((C,), jnp.int32),
            pltpu.VMEM((C,), jnp.int32),
            pltpu.VMEM((C,), jnp.int32),
            pltpu.VMEM((C,), jnp.int32),
            pltpu.VMEM((C,), jnp.int32),
            pltpu.VMEM((C, 2 * HW), jnp.float32),
            pltpu.VMEM((C, HW), jnp.float32),
            pltpu.VMEM((C, 2 * HW), jnp.float32),
            pltpu.VMEM((C, HW), jnp.float32),
            pltpu.VMEM((G, 16, ROWW), jnp.float32),
            pltpu.VMEM_SHARED((N, ROWW), jnp.float32),
            pltpu.SemaphoreType.DMA,
            pltpu.SemaphoreType.DMA,
            pltpu.SemaphoreType.DMA,
            pltpu.SemaphoreType.DMA,
            pltpu.SemaphoreType.DMA,
        ],
        compiler_params=pltpu.CompilerParams(use_tc_tiling_on_sc=False,
                                             needs_layout_passes=False),
    )
    return kern(kv, q, src, dst)


def _combine_body(p_ref, out_ref):
    cols = []
    for c in range(NC):
        a = p_ref[c]
        for h in range(HH):
            wv = a[:, h * D:(h + 1) * D]
            z = a[:, HW + h:HW + h + 1]
            cols.append(wv / z)
    out_ref[...] = jnp.concatenate(cols, axis=1)


def _combine(parts):
    blk = 1000
    return pl.pallas_call(
        _combine_body,
        grid=(N // blk,),
        in_specs=[pl.BlockSpec((NC, blk, ROWW), lambda i: (0, i, 0))],
        out_specs=pl.BlockSpec((blk, HD), lambda i: (i, 0)),
        out_shape=jax.ShapeDtypeStruct((N, HD), jnp.float32),
    )(parts)


def kernel(state, edge_index, Wq, bq, Wk, bk, Wv, bv):
    q, kv = _project(state, Wq, bq, Wk, bk, Wv, bv)
    src = edge_index[0]
    dst = edge_index[1]
    parts = _edge_phase(kv, q, src, dst)
    out = _combine(parts)
    return out.reshape(N, H, D)


# speedup vs baseline: 95.9618x; 6.0284x over previous
import jax
import jax.numpy as jnp
from jax import lax
from jax.experimental import pallas as pl
from jax.experimental.pallas import tpu as pltpu
from jax.experimental.pallas import tpu_sc as plsc

N = 10000
E = 320000
IN_DIM = 128
H = 8
D = 16
HD = H * D
HH = H // 2
HW = HH * D

NC = 2
NS = 16
EPT = E // NS
C = 80
G = C // 16
NCHUNK = EPT // C
NPAIR = NCHUNK // 2
ROWW = 80
RPT = N // NS


def _proj_body(x_ref, wq_ref, bq_ref, wk_ref, bk_ref, wv_ref, bv_ref,
               qt_ref, kvt_ref):
    x = x_ref[...]
    scale = 1.0 / (D ** 0.5)
    q = lax.dot_general(x, wq_ref[...], (((1,), (0,)), ((), ())),
                        preferred_element_type=jnp.float32)
    k = lax.dot_general(x, wk_ref[...], (((1,), (0,)), ((), ())),
                        preferred_element_type=jnp.float32)
    v = lax.dot_general(x, wv_ref[...], (((1,), (0,)), ((), ())),
                        preferred_element_type=jnp.float32)
    q = (q + bq_ref[...]) * scale
    k = k + bk_ref[...]
    v = v + bv_ref[...]
    for c in range(NC):
        qt_ref[c] = q[:, c * HW:(c + 1) * HW]
        kvt_ref[c, :, :HW] = k[:, c * HW:(c + 1) * HW]
        kvt_ref[c, :, HW:] = v[:, c * HW:(c + 1) * HW]


def _project(state, Wq, bq, Wk, bk, Wv, bv):
    blk = 1000
    grid = (N // blk,)
    w_spec = pl.BlockSpec((IN_DIM, HD), lambda i: (0, 0))
    b_spec = pl.BlockSpec((1, HD), lambda i: (0, 0))
    x_spec = pl.BlockSpec((blk, IN_DIM), lambda i: (i, 0))
    qt, kvt = pl.pallas_call(
        _proj_body,
        grid=grid,
        in_specs=[x_spec, w_spec, b_spec, w_spec, b_spec, w_spec, b_spec],
        out_specs=[pl.BlockSpec((NC, blk, HW), lambda i: (0, i, 0)),
                   pl.BlockSpec((NC, blk, 2 * HW), lambda i: (0, i, 0))],
        out_shape=[jax.ShapeDtypeStruct((NC, N, HW), jnp.float32),
                   jax.ShapeDtypeStruct((NC, N, 2 * HW), jnp.float32)],
    )(state, Wq, bq.reshape(1, HD), Wk, bk.reshape(1, HD),
      Wv, bv.reshape(1, HD))
    return qt.reshape(NC * N, HW), kvt.reshape(NC * N, 2 * HW)


def _edge_body(kv_hbm, q_hbm, src_hbm, dst_hbm, out_hbm,
               isA, idA, iqA, isB, idB, iqB, sdA, sdB,
               kvgA, qgA, kvgB, qgB, wrowA, wrowB, acc,
               sem_iA, sem_iB, sem_gA, sem_gB, sem_sc):
    cid = lax.axis_index("c")
    sid = lax.axis_index("s")
    cbase = cid * N
    liota = lax.iota(jnp.int32, 16)
    zeros16 = jnp.zeros((16,), jnp.float32)
    full15 = jnp.full((16,), 15, jnp.int32)
    m4 = liota < HH

    def _zfill(j, _):
        wrowA[j // (ROWW // 16), pl.ds((j % (ROWW // 16)) * 16, 16)] = zeros16
        return 0
    lax.fori_loop(0, C * (ROWW // 16), _zfill, 0)

    def _zcopy(t, _):
        pltpu.sync_copy(wrowA, acc.at[pl.ds(sid * RPT + t * C, C)])
        return 0
    lax.fori_loop(0, RPT // C, _zcopy, 0)
    pltpu.sync_copy(wrowA.at[pl.ds(0, RPT % C)],
                    acc.at[pl.ds(sid * RPT + (RPT // C) * C, RPT % C)])

    plsc.subcore_barrier()

    ebase = sid * EPT

    def issue_idx(c, is_buf, id_buf, sem):
        base = ebase + c * C
        pltpu.async_copy(src_hbm.at[pl.ds(base, C)], is_buf, sem)
        pltpu.async_copy(dst_hbm.at[pl.ds(base, C)], id_buf, sem)

    def drain_idx(sem, is_buf, id_buf):
        pltpu.make_async_copy(src_hbm.at[pl.ds(0, C)], is_buf, sem).wait()
        pltpu.make_async_copy(src_hbm.at[pl.ds(0, C)], id_buf, sem).wait()

    def adjust_idx(is_buf, id_buf, iq_buf):
        for j in range(G):
            sl = pl.ds(j * 16, 16)
            is_buf[sl] = is_buf[sl] + cbase
            iq_buf[sl] = id_buf[sl] + cbase

    def issue_gath(is_buf, iq_buf, kvg, qg, sem):
        pltpu.async_copy(kv_hbm.at[is_buf], kvg, sem)
        pltpu.async_copy(q_hbm.at[iq_buf], qg, sem)

    def drain_gath(sem, kvg, qg):
        pltpu.make_async_copy(kv_hbm.at[pl.ds(0, C)], kvg, sem).wait()
        pltpu.make_async_copy(q_hbm.at[pl.ds(0, C)], qg, sem).wait()

    gdn = lax.GatherDimensionNumbers(offset_dims=(), collapsed_slice_dims=(0,),
                                     start_index_map=(0,))

    def _bcast(x, lane):
        idx = jnp.full((16, 1), lane, jnp.int32)
        return lax.gather(x, idx, gdn, slice_sizes=(1,),
                          mode=lax.GatherScatterMode.PROMISE_IN_BOUNDS)

    def comp(kvg, qg, id_buf, sd_buf, wrow):
        @plsc.parallel_loop(0, C, unroll=4)
        def _edge(e):
            sigma = zeros16
            for h in range(HH):
                k = kvg[e, pl.ds(h * D, D)]
                q = qg[e, pl.ds(h * D, D)]
                cum = plsc.cumsum(k * q)
                sigma = jnp.where(liota == h, _bcast(cum, 15), sigma)
            sigma = jnp.exp(jnp.clip(sigma, -5.0, 5.0))
            sigma = jnp.where(m4, sigma, 0.0)
            for h in range(HH):
                v = kvg[e, pl.ds(HW + h * D, D)]
                wrow[e, pl.ds(h * D, D)] = v * _bcast(sigma, h)
            wrow[e, pl.ds(HW, 16)] = sigma

        for j in range(C // 16):
            sl = pl.ds(j * 16, 16)
            sd_buf[sl] = id_buf[sl]
        pltpu.async_copy(wrow, acc.at[sd_buf], sem_sc, add=True)

    def drain_sc():
        pltpu.make_async_copy(out_hbm.at[0, pl.ds(0, C)], wrowA,
                              sem_sc).wait()

    issue_idx(0, isA, idA, sem_iA)
    drain_idx(sem_iA, isA, idA)
    adjust_idx(isA, idA, iqA)
    issue_gath(isA, iqA, kvgA, qgA, sem_gA)
    issue_idx(1, isB, idB, sem_iB)

    def pair_body(t, _):
        a = 2 * t
        b = 2 * t + 1
        drain_gath(sem_gA, kvgA, qgA)
        drain_idx(sem_iB, isB, idB)
        adjust_idx(isB, idB, iqB)
        issue_gath(isB, iqB, kvgB, qgB, sem_gB)

        @pl.when(t > 0)
        def _():
            drain_sc()

        comp(kvgA, qgA, idA, sdA, wrowA)

        @pl.when(t < NPAIR - 1)
        def _():
            issue_idx(a + 2, isA, idA, sem_iA)

        drain_gath(sem_gB, kvgB, qgB)

        @pl.when(t < NPAIR - 1)
        def _():
            drain_idx(sem_iA, isA, idA)
            adjust_idx(isA, idA, iqA)
            issue_gath(isA, iqA, kvgA, qgA, sem_gA)

        @pl.when(t > 0)
        def _():
            drain_sc()

        comp(kvgB, qgB, idB, sdB, wrowB)

        @pl.when(t < NPAIR - 1)
        def _():
            issue_idx(b + 2, isB, idB, sem_iB)

        return 0

    lax.fori_loop(0, NPAIR, pair_body, 0)

    drain_sc()
    drain_sc()
    plsc.subcore_barrier()

    pltpu.sync_copy(acc.at[pl.ds(sid * RPT, RPT)],
                    out_hbm.at[cid, pl.ds(sid * RPT, RPT)])


def _edge_phase(kv, q, src, dst):
    mesh = plsc.VectorSubcoreMesh(core_axis_name="c", subcore_axis_name="s")
    kern = pl.kernel(
        _edge_body,
        out_type=jax.ShapeDtypeStruct((NC, N, ROWW), jnp.float32),
        mesh=mesh,
        scratch_types=[
            pltpu.VMEM((C,), jnp.int32),
            pltpu.VMEM((C,), jnp.int32),
            pltpu.VMEM((C,), jnp.int32),
            pltpu.VMEM((C,), jnp.int32),
            pltpu.VMEM((C,), jnp.int32),
            pltpu.VMEM((C,), jnp.int32),
            pltpu.VMEM((C,), jnp.int32),
            pltpu.VMEM((C,), jnp.int32),
            pltpu.VMEM((C, 2 * HW), jnp.float32),
            pltpu.VMEM((C, HW), jnp.float32),
            pltpu.VMEM((C, 2 * HW), jnp.float32),
            pltpu.VMEM((C, HW), jnp.float32),
            pltpu.VMEM((C, ROWW), jnp.float32),
            pltpu.VMEM((C, ROWW), jnp.float32),
            pltpu.VMEM_SHARED((N, ROWW), jnp.float32),
            pltpu.SemaphoreType.DMA,
            pltpu.SemaphoreType.DMA,
            pltpu.SemaphoreType.DMA,
            pltpu.SemaphoreType.DMA,
            pltpu.SemaphoreType.DMA,
        ],
        compiler_params=pltpu.CompilerParams(use_tc_tiling_on_sc=False,
                                             needs_layout_passes=False),
    )
    return kern(kv, q, src, dst)


def _combine_body(p_ref, out_ref):
    cols = []
    for c in range(NC):
        a = p_ref[c]
        for h in range(HH):
            wv = a[:, h * D:(h + 1) * D]
            z = a[:, HW + h:HW + h + 1]
            cols.append(wv / z)
    out_ref[...] = jnp.concatenate(cols, axis=1)


def _combine(parts):
    blk = 1000
    return pl.pallas_call(
        _combine_body,
        grid=(N // blk,),
        in_specs=[pl.BlockSpec((NC, blk, ROWW), lambda i: (0, i, 0))],
        out_specs=pl.BlockSpec((blk, HD), lambda i: (i, 0)),
        out_shape=jax.ShapeDtypeStruct((N, HD), jnp.float32),
    )(parts)


def kernel(state, edge_index, Wq, bq, Wk, bk, Wv, bv):
    q, kv = _project(state, Wq, bq, Wk, bk, Wv, bv)
    src = edge_index[0]
    dst = edge_index[1]
    parts = _edge_phase(kv, q, src, dst)
    out = _combine(parts)
    return out.reshape(N, H, D)
